# baseline (speedup 1.0000x reference)
import functools

import jax
import jax.numpy as jnp
from jax import lax
from jax.experimental import pallas as pl
from jax.experimental.pallas import tpu as pltpu
from jax.experimental.pallas import tpu_sc as plsc

N_NODES = 10000
N_EDGES = 320000
D = 128

NC = 2
NS = 16
NW = NC * NS
CH = 64
EPW = 10240
NCHUNK = EPW // CH
IBLK = 32
NBLK = NCHUNK // IBLK
NPAD = 10240
TRASH = N_NODES + 8
SLICE = NPAD // NS
SUB = 128


def _tables_body(feat_ref, w_ref, b_ref, t_ref):
    w1 = w_ref[:, :D]
    w2 = w_ref[:, D:]
    f = feat_ref[...]
    dn = (((1,), (1,)), ((), ()))
    t_ref[:N_NODES] = lax.dot_general(
        f, w1 - w2, dn, preferred_element_type=jnp.float32) + b_ref[...]
    t_ref[N_NODES:] = lax.dot_general(
        f, w2, dn, preferred_element_type=jnp.float32)


def _make_tables(feature, W, b):
    return pl.pallas_call(
        _tables_body,
        out_shape=jax.ShapeDtypeStruct((2 * N_NODES, D), jnp.float32),
    )(feature, W, b.reshape(1, D))


def _edge_body(t_hbm, comb_hbm, dst_hbm, psum_hbm, pcnt_hbm,
               idx_comb, idx_dst, buf, hist, acc, sem):
    c = lax.axis_index("c")
    s = lax.axis_index("s")
    w = c * NS + s

    zeros16 = jnp.zeros((16,), jnp.float32)
    e0 = jnp.where(lax.iota(jnp.int32, 16) == 0, 1.0, 0.0)

    def _fill_buf(i, _):
        for j in range(D // 16):
            buf[i, pl.ds(j * 16, 16)] = zeros16
        return 0
    lax.fori_loop(0, SUB, _fill_buf, 0)

    def _fill_hist(i, _):
        hist[pl.ds(i * 16, 16)] = zeros16
        return 0
    lax.fori_loop(0, NPAD // 16, _fill_hist, 0)

    base = s * SLICE
    for k in range(SLICE // SUB):
        pltpu.sync_copy(buf.at[pl.ds(0, SUB)],
                        acc.at[pl.ds(base + k * SUB, SUB)])
    plsc.subcore_barrier()

    def _chunk(ci, _):
        pltpu.async_copy(t_hbm.at[idx_comb.at[ci]], buf, sem)
        pltpu.make_async_copy(t_hbm.at[pl.ds(0, 2 * CH)], buf, sem).wait()

        def _row(i, _):
            for j in range(D // 16):
                sl = pl.ds(j * 16, 16)
                buf[i, sl] = jnp.maximum(buf[i, sl] + buf[CH + i, sl], 0.0)
            return 0
        lax.fori_loop(0, CH, _row, 0)

        def _cnt(k2, _):
            idxv = idx_dst[ci, pl.ds(k2 * 16, 16)]
            for l in range(16):
                hsl = pl.ds(idxv[l], 16)
                hist[hsl] = hist[hsl] + e0
            return 0
        lax.fori_loop(0, CH // 16, _cnt, 0)

        pltpu.sync_copy(buf.at[pl.ds(0, CH)], acc.at[idx_dst.at[ci]],
                        add=True)
        return 0

    for bi in range(NBLK):
        pltpu.sync_copy(comb_hbm.at[w, bi], idx_comb)
        pltpu.sync_copy(dst_hbm.at[w, bi], idx_dst)
        lax.fori_loop(0, IBLK, _chunk, 0)

    plsc.subcore_barrier()

    for k in range(SLICE // SUB):
        off = base + k * SUB
        pltpu.sync_copy(acc.at[pl.ds(off, SUB)], buf.at[pl.ds(0, SUB)])
        pltpu.sync_copy(buf.at[pl.ds(0, SUB)], psum_hbm.at[c, pl.ds(off, SUB)])
    pltpu.sync_copy(hist, pcnt_hbm.at[w])


@functools.partial(
    pl.kernel,
    out_type=(
        jax.ShapeDtypeStruct((NC, NPAD, D), jnp.float32),
        jax.ShapeDtypeStruct((NW, NPAD), jnp.float32),
    ),
    mesh=plsc.VectorSubcoreMesh(core_axis_name="c", subcore_axis_name="s"),
    scratch_types=[
        pltpu.VMEM((IBLK, 2 * CH), jnp.int32),
        pltpu.VMEM((IBLK, CH), jnp.int32),
        pltpu.VMEM((2 * CH, D), jnp.float32),
        pltpu.VMEM((NPAD,), jnp.float32),
        pltpu.VMEM_SHARED((NPAD, D), jnp.float32),
        pltpu.SemaphoreType.DMA,
    ],
)
def _edge_kernel(t_hbm, comb_hbm, dst_hbm, psum_hbm, pcnt_hbm,
                 idx_comb, idx_dst, buf, hist, acc, sem):
    _edge_body(t_hbm, comb_hbm, dst_hbm, psum_hbm, pcnt_hbm,
               idx_comb, idx_dst, buf, hist, acc, sem)


def _final_body(psum_ref, pcnt_ref, out_ref):
    tot = psum_ref[0, :N_NODES] + psum_ref[1, :N_NODES]
    cnt = jnp.sum(pcnt_ref[...], axis=0)
    cntcol = cnt[:N_NODES].reshape(N_NODES, 1)
    out_ref[...] = tot / jnp.maximum(cntcol, 1.0)


def _finalize(psum, pcnt):
    return pl.pallas_call(
        _final_body,
        out_shape=jax.ShapeDtypeStruct((N_NODES, D), jnp.float32),
    )(psum, pcnt)


def kernel(feature, edge_index, W, b):
    t_tab = _make_tables(feature, W, b)
    npad_e = NW * EPW - N_EDGES
    src = jnp.concatenate(
        [edge_index[0], jnp.zeros((npad_e,), jnp.int32)])
    dst = jnp.concatenate(
        [edge_index[1], jnp.full((npad_e,), TRASH, jnp.int32)])
    dst2 = dst.reshape(-1, CH)
    src2 = src.reshape(-1, CH)
    comb = jnp.concatenate([dst2, src2 + N_NODES], axis=1)
    comb4 = comb.reshape(NW, NBLK, IBLK, 2 * CH)
    dst4 = dst2.reshape(NW, NBLK, IBLK, CH)
    psum, pcnt = _edge_kernel(t_tab, comb4, dst4)
    return _finalize(psum, pcnt)


# speedup vs baseline: 2.0764x; 2.0764x over previous
import functools

import jax
import jax.numpy as jnp
from jax import lax
from jax.experimental import pallas as pl
from jax.experimental.pallas import tpu as pltpu
from jax.experimental.pallas import tpu_sc as plsc

N_NODES = 10000
N_EDGES = 320000
D = 128

NC = 2
NS = 16
NW = NC * NS
CH = 80
HC = CH // 2
EPW = N_EDGES // NW
NCHUNK = EPW // CH
IBLK = 25
NBLK = NCHUNK // IBLK
NPAD = 10240
SLICE = NPAD // NS
SUB = CH


def _tables_body(feat_ref, w_ref, b_ref, a_ref, bt_ref):
    w1 = w_ref[:, :D]
    w2 = w_ref[:, D:]
    f = feat_ref[...]
    dn = (((1,), (1,)), ((), ()))
    a_ref[...] = lax.dot_general(f, w1 - w2, dn,
                                 preferred_element_type=jnp.float32) + b_ref[...]
    bt_ref[...] = lax.dot_general(f, w2, dn,
                                  preferred_element_type=jnp.float32)


def _make_tables(feature, W, b):
    return pl.pallas_call(
        _tables_body,
        out_shape=(
            jax.ShapeDtypeStruct((N_NODES, D), jnp.float32),
            jax.ShapeDtypeStruct((N_NODES, D), jnp.float32),
        ),
    )(feature, W, b.reshape(1, D))


def _edge_body(a_hbm, b_hbm, src_hbm, dst_hbm, psum_hbm, pcnt_hbm,
               idx_src, idx_dst, buf_a, buf_b, hist, acc,
               sem0, sem1, sem2, sem3):
    c = lax.axis_index("c")
    s = lax.axis_index("s")
    w = c * NS + s

    zeros16 = jnp.zeros((16,), jnp.float32)
    e0 = jnp.where(lax.iota(jnp.int32, 16) == 0, 1.0, 0.0)

    def _fill_buf(i, _):
        for j in range(D // 16):
            buf_a[i, pl.ds(j * 16, 16)] = zeros16
        return 0
    lax.fori_loop(0, CH, _fill_buf, 0)

    def _fill_hist(i, _):
        hist[pl.ds(i * 16, 16)] = zeros16
        return 0
    lax.fori_loop(0, NPAD // 16, _fill_hist, 0)

    base = s * SLICE
    for k in range(SLICE // SUB):
        pltpu.sync_copy(buf_a, acc.at[pl.ds(base + k * SUB, SUB)])
    plsc.subcore_barrier()

    def _chunk(ci, _):
        pltpu.async_copy(a_hbm.at[idx_dst.at[ci, pl.ds(0, HC)]],
                         buf_a.at[pl.ds(0, HC)], sem0)
        pltpu.async_copy(a_hbm.at[idx_dst.at[ci, pl.ds(HC, HC)]],
                         buf_a.at[pl.ds(HC, HC)], sem1)
        pltpu.async_copy(b_hbm.at[idx_src.at[ci, pl.ds(0, HC)]],
                         buf_b.at[pl.ds(0, HC)], sem2)
        pltpu.async_copy(b_hbm.at[idx_src.at[ci, pl.ds(HC, HC)]],
                         buf_b.at[pl.ds(HC, HC)], sem3)
        pltpu.make_async_copy(a_hbm.at[pl.ds(0, HC)],
                              buf_a.at[pl.ds(0, HC)], sem0).wait()
        pltpu.make_async_copy(a_hbm.at[pl.ds(0, HC)],
                              buf_a.at[pl.ds(HC, HC)], sem1).wait()
        pltpu.make_async_copy(b_hbm.at[pl.ds(0, HC)],
                              buf_b.at[pl.ds(0, HC)], sem2).wait()
        pltpu.make_async_copy(b_hbm.at[pl.ds(0, HC)],
                              buf_b.at[pl.ds(HC, HC)], sem3).wait()

        def _row(i, _):
            for j in range(D // 16):
                sl = pl.ds(j * 16, 16)
                buf_a[i, sl] = jnp.maximum(buf_a[i, sl] + buf_b[i, sl], 0.0)
            return 0
        lax.fori_loop(0, CH, _row, 0)

        def _cnt(k2, _):
            idxv = idx_dst[ci, pl.ds(k2 * 16, 16)]
            for l in range(16):
                hsl = pl.ds(idxv[l], 16)
                hist[hsl] = hist[hsl] + e0
            return 0
        lax.fori_loop(0, CH // 16, _cnt, 0)

        pltpu.sync_copy(buf_a, acc.at[idx_dst.at[ci]], add=True)
        return 0

    for bi in range(NBLK):
        pltpu.sync_copy(src_hbm.at[w, bi], idx_src)
        pltpu.sync_copy(dst_hbm.at[w, bi], idx_dst)
        lax.fori_loop(0, IBLK, _chunk, 0)

    plsc.subcore_barrier()

    for k in range(SLICE // SUB):
        off = base + k * SUB
        pltpu.sync_copy(acc.at[pl.ds(off, SUB)], buf_a)
        pltpu.sync_copy(buf_a, psum_hbm.at[c, pl.ds(off, SUB)])
    pltpu.sync_copy(hist, pcnt_hbm.at[w])


@functools.partial(
    pl.kernel,
    out_type=(
        jax.ShapeDtypeStruct((NC, NPAD, D), jnp.float32),
        jax.ShapeDtypeStruct((NW, NPAD), jnp.float32),
    ),
    mesh=plsc.VectorSubcoreMesh(core_axis_name="c", subcore_axis_name="s"),
    scratch_types=[
        pltpu.VMEM((IBLK, CH), jnp.int32),
        pltpu.VMEM((IBLK, CH), jnp.int32),
        pltpu.VMEM((CH, D), jnp.float32),
        pltpu.VMEM((CH, D), jnp.float32),
        pltpu.VMEM((NPAD,), jnp.float32),
        pltpu.VMEM_SHARED((NPAD, D), jnp.float32),
        pltpu.SemaphoreType.DMA,
        pltpu.SemaphoreType.DMA,
        pltpu.SemaphoreType.DMA,
        pltpu.SemaphoreType.DMA,
    ],
)
def _edge_kernel(a_hbm, b_hbm, src_hbm, dst_hbm, psum_hbm, pcnt_hbm,
                 idx_src, idx_dst, buf_a, buf_b, hist, acc,
                 sem0, sem1, sem2, sem3):
    _edge_body(a_hbm, b_hbm, src_hbm, dst_hbm, psum_hbm, pcnt_hbm,
               idx_src, idx_dst, buf_a, buf_b, hist, acc,
               sem0, sem1, sem2, sem3)


def _final_body(psum_ref, pcnt_ref, out_ref):
    tot = psum_ref[0, :N_NODES] + psum_ref[1, :N_NODES]
    cnt = jnp.sum(pcnt_ref[...], axis=0)
    cntcol = cnt[:N_NODES].reshape(N_NODES, 1)
    out_ref[...] = tot / jnp.maximum(cntcol, 1.0)


def _finalize(psum, pcnt):
    return pl.pallas_call(
        _final_body,
        out_shape=jax.ShapeDtypeStruct((N_NODES, D), jnp.float32),
    )(psum, pcnt)


def kernel(feature, edge_index, W, b):
    a_tab, b_tab = _make_tables(feature, W, b)
    src4 = edge_index[0].reshape(NW, NBLK, IBLK, CH)
    dst4 = edge_index[1].reshape(NW, NBLK, IBLK, CH)
    psum, pcnt = _edge_kernel(a_tab, b_tab, src4, dst4)
    return _finalize(psum, pcnt)


# speedup vs baseline: 2.5770x; 1.2411x over previous
import functools

import jax
import jax.numpy as jnp
from jax import lax
from jax.experimental import pallas as pl
from jax.experimental.pallas import tpu as pltpu
from jax.experimental.pallas import tpu_sc as plsc

N_NODES = 10000
N_EDGES = 320000
D = 128

NC = 2
NS = 16
NW = NC * NS
CH = 80
HC = CH // 2
EPW = N_EDGES // NW
NCHUNK = EPW // CH
IBLK = 25
NBLK = NCHUNK // IBLK
NPAD = 10240
SLICE = NPAD // NS
SUB = CH


def _tables_body(feat_ref, w_ref, b_ref, a_ref, bt_ref):
    w1 = w_ref[:, :D]
    w2 = w_ref[:, D:]
    f = feat_ref[...]
    dn = (((1,), (1,)), ((), ()))
    a_ref[...] = lax.dot_general(f, w1 - w2, dn,
                                 preferred_element_type=jnp.float32) + b_ref[...]
    bt_ref[...] = lax.dot_general(f, w2, dn,
                                  preferred_element_type=jnp.float32)


def _make_tables(feature, W, b):
    return pl.pallas_call(
        _tables_body,
        out_shape=(
            jax.ShapeDtypeStruct((N_NODES, D), jnp.float32),
            jax.ShapeDtypeStruct((N_NODES, D), jnp.float32),
        ),
    )(feature, W, b.reshape(1, D))


def _edge_body(a_hbm, b_hbm, src_hbm, dst_hbm, psum_hbm, pcnt_hbm,
               idx_src, idx_dst, buf_a, buf_b, hist, acc,
               sem0, sem1, sem_s):
    c = lax.axis_index("c")
    s = lax.axis_index("s")
    w = c * NS + s

    zeros16 = jnp.zeros((16,), jnp.float32)
    e0 = jnp.where(lax.iota(jnp.int32, 16) == 0, 1.0, 0.0)

    def _fill_buf(i, _):
        for j in range(D // 16):
            buf_a[i, pl.ds(j * 16, 16)] = zeros16
        return 0
    lax.fori_loop(0, CH, _fill_buf, 0)

    def _fill_hist(i, _):
        hist[pl.ds(i * 16, 16)] = zeros16
        return 0
    lax.fori_loop(0, NPAD // 16, _fill_hist, 0)

    base = s * SLICE
    for k in range(SLICE // SUB):
        pltpu.sync_copy(buf_a, acc.at[pl.ds(base + k * SUB, SUB)])
    plsc.subcore_barrier()

    def _chunk(ci, _):
        pltpu.make_async_copy(a_hbm.at[pl.ds(0, CH)], buf_a, sem0).wait()
        pltpu.make_async_copy(b_hbm.at[pl.ds(0, CH)], buf_b, sem1).wait()

        def _row(i, _):
            for j in range(D // 16):
                sl = pl.ds(j * 16, 16)
                buf_b[i, sl] = jnp.maximum(buf_a[i, sl] + buf_b[i, sl], 0.0)
            return 0
        lax.fori_loop(0, CH, _row, 0)

        @pl.when(ci + 1 < IBLK)
        def _prefetch_a():
            pltpu.async_copy(a_hbm.at[idx_dst.at[ci + 1]], buf_a, sem0)

        pltpu.async_copy(buf_b, acc.at[idx_dst.at[ci]], sem_s, add=True)

        def _cnt(k2, _):
            idxv = idx_dst[ci, pl.ds(k2 * 16, 16)]
            for l in range(16):
                hsl = pl.ds(idxv[l], 16)
                hist[hsl] = hist[hsl] + e0
            return 0
        lax.fori_loop(0, CH // 16, _cnt, 0)

        pltpu.make_async_copy(buf_b, acc.at[idx_dst.at[ci]], sem_s).wait()

        @pl.when(ci + 1 < IBLK)
        def _prefetch_b():
            pltpu.async_copy(b_hbm.at[idx_src.at[ci + 1]], buf_b, sem1)
        return 0

    for bi in range(NBLK):
        pltpu.sync_copy(src_hbm.at[w, bi], idx_src)
        pltpu.sync_copy(dst_hbm.at[w, bi], idx_dst)
        pltpu.async_copy(a_hbm.at[idx_dst.at[0]], buf_a, sem0)
        pltpu.async_copy(b_hbm.at[idx_src.at[0]], buf_b, sem1)
        lax.fori_loop(0, IBLK, _chunk, 0)

    plsc.subcore_barrier()

    for k in range(SLICE // SUB):
        off = base + k * SUB
        pltpu.sync_copy(acc.at[pl.ds(off, SUB)], buf_a)
        pltpu.sync_copy(buf_a, psum_hbm.at[c, pl.ds(off, SUB)])
    pltpu.sync_copy(hist, pcnt_hbm.at[w])


@functools.partial(
    pl.kernel,
    out_type=(
        jax.ShapeDtypeStruct((NC, NPAD, D), jnp.float32),
        jax.ShapeDtypeStruct((NW, NPAD), jnp.float32),
    ),
    mesh=plsc.VectorSubcoreMesh(core_axis_name="c", subcore_axis_name="s"),
    scratch_types=[
        pltpu.VMEM((IBLK, CH), jnp.int32),
        pltpu.VMEM((IBLK, CH), jnp.int32),
        pltpu.VMEM((CH, D), jnp.float32),
        pltpu.VMEM((CH, D), jnp.float32),
        pltpu.VMEM((NPAD,), jnp.float32),
        pltpu.VMEM_SHARED((NPAD, D), jnp.float32),
        pltpu.SemaphoreType.DMA,
        pltpu.SemaphoreType.DMA,
        pltpu.SemaphoreType.DMA,
    ],
)
def _edge_kernel(a_hbm, b_hbm, src_hbm, dst_hbm, psum_hbm, pcnt_hbm,
                 idx_src, idx_dst, buf_a, buf_b, hist, acc,
                 sem0, sem1, sem_s):
    _edge_body(a_hbm, b_hbm, src_hbm, dst_hbm, psum_hbm, pcnt_hbm,
               idx_src, idx_dst, buf_a, buf_b, hist, acc,
               sem0, sem1, sem_s)


def _final_body(psum_ref, pcnt_ref, out_ref):
    tot = psum_ref[0, :N_NODES] + psum_ref[1, :N_NODES]
    cnt = jnp.sum(pcnt_ref[...], axis=0)
    cntcol = cnt[:N_NODES].reshape(N_NODES, 1)
    out_ref[...] = tot / jnp.maximum(cntcol, 1.0)


def _finalize(psum, pcnt):
    return pl.pallas_call(
        _final_body,
        out_shape=jax.ShapeDtypeStruct((N_NODES, D), jnp.float32),
    )(psum, pcnt)


def kernel(feature, edge_index, W, b):
    a_tab, b_tab = _make_tables(feature, W, b)
    src4 = edge_index[0].reshape(NW, NBLK, IBLK, CH)
    dst4 = edge_index[1].reshape(NW, NBLK, IBLK, CH)
    psum, pcnt = _edge_kernel(a_tab, b_tab, src4, dst4)
    return _finalize(psum, pcnt)
